# Initial kernel scaffold; baseline (speedup 1.0000x reference)
import jax
import jax.numpy as jnp
from jax.experimental import pallas as pl


def kernel(x, indices):
    raise NotImplementedError("write your pallas kernel here")


# speedup vs baseline: 59.0809x; 59.0809x over previous
import functools

import jax
import jax.numpy as jnp
from jax import lax
from jax.experimental import pallas as pl
from jax.experimental.pallas import tpu as pltpu
from jax.experimental.pallas import tpu_sc as plsc

_B, _C, _H, _W = 2, 96, 112, 112
_HW = _H * _W
_OHW = 4 * _HW
_P = _B * _C
_NW = 32
_PPW = _P // _NW
_VECS = _HW // 16
_ZVECS = _OHW // 16

_mesh = plsc.VectorSubcoreMesh(core_axis_name="c", subcore_axis_name="s")


@functools.partial(
    pl.kernel,
    out_type=jax.ShapeDtypeStruct((_P, _OHW), jnp.float32),
    mesh=_mesh,
    scratch_types=[
        pltpu.VMEM((_HW,), jnp.float32),
        pltpu.VMEM((_HW,), jnp.int32),
        pltpu.VMEM((_OHW,), jnp.float32),
    ],
    compiler_params=pltpu.CompilerParams(needs_layout_passes=False),
)
def _unpool(x_hbm, idx_hbm, out_hbm, xbuf, idxbuf, outbuf):
    wid = lax.axis_index("s") * 2 + lax.axis_index("c")
    zeros = jnp.zeros((16,), jnp.float32)

    def plane_body(p, carry):
        plane = wid * _PPW + p
        pltpu.sync_copy(x_hbm.at[plane], xbuf)
        pltpu.sync_copy(idx_hbm.at[plane], idxbuf)

        def zbody(j, c):
            outbuf[pl.ds(j * 16, 16)] = zeros
            return c

        lax.fori_loop(0, _ZVECS, zbody, 0, unroll=8)

        def sbody(k, c):
            idxv = idxbuf[pl.ds(k * 16, 16)]
            xv = xbuf[pl.ds(k * 16, 16)]
            plsc.store_scatter(outbuf, [idxv], xv)
            return c

        lax.fori_loop(0, _VECS, sbody, 0)
        pltpu.sync_copy(outbuf, out_hbm.at[plane])
        return carry

    lax.fori_loop(0, _PPW, plane_body, 0)


def kernel(x, indices):
    x2 = x.reshape(_P, _HW)
    idx2 = indices.reshape(_P, _HW).astype(jnp.int32)
    out = _unpool(x2, idx2)
    return out.reshape(_B, _C, 2 * _H, 2 * _W)
